# Initial kernel scaffold; baseline (speedup 1.0000x reference)
import jax
import jax.numpy as jnp
from jax.experimental import pallas as pl


def kernel(x, W1, W2, W3, W4, W5, gamma1, beta1, gamma2, beta2, gamma3, beta3, gamma4, beta4, gamma5, beta5):
    raise NotImplementedError("write your pallas kernel here")


# speedup vs baseline: 10.3516x; 10.3516x over previous
import functools

import jax
import jax.numpy as jnp
from jax import lax
from jax.experimental import pallas as pl
from jax.experimental.pallas import tpu as pltpu
from jax.experimental.pallas import tpu_sc as plsc

B = 16
N = 2048
K = 20
RB = 256
NB = N // RB
EPS = 1e-5
NEG = -1e30

SC_CORES = 2
SC_SUBCORES = 16
NW = SC_CORES * SC_SUBCORES
TOTAL_ROWS = B * K * N
ROWS_PER_W = TOTAL_ROWS // NW
GCHUNK = 128
NCHUNK = ROWS_PER_W // GCHUNK


def _leaky(z):
    return jnp.where(z >= 0, z, 0.2 * z)


def _knn_body(xall_ref, xrow_ref, idx_ref):
    b = pl.program_id(0)
    xa = xall_ref[0]
    xb = xrow_ref[0]
    cross = lax.dot_general(xa.astype(jnp.bfloat16), xb.astype(jnp.bfloat16),
                            (((1,), (1,)), ((), ())),
                            preferred_element_type=jnp.float32)
    na2 = jnp.sum(xa * xa, axis=1, keepdims=True)
    key = 2.0 * cross - na2
    rows = lax.broadcasted_iota(jnp.int32, (N, RB), 0)
    sels = []
    for _ in range(K):
        m = jnp.max(key, axis=0, keepdims=True)
        sel = jnp.min(jnp.where(key == m, rows, N), axis=0, keepdims=True)
        sels.append(sel)
        key = jnp.where(rows == sel, NEG, key)
    idx_ref[0] = jnp.concatenate(sels, axis=0) + b * N


def _knn(xT):
    C = xT.shape[2]
    return pl.pallas_call(
        _knn_body,
        grid=(B, NB),
        in_specs=[
            pl.BlockSpec((1, N, C), lambda b, r: (b, 0, 0)),
            pl.BlockSpec((1, RB, C), lambda b, r: (b, r, 0)),
        ],
        out_specs=pl.BlockSpec((1, K, RB), lambda b, r: (b, 0, r)),
        out_shape=jax.ShapeDtypeStruct((B, K, N), jnp.int32),
    )(xT, xT)


def _sc_gather_call(table, ids, Cp):
    mesh = plsc.VectorSubcoreMesh(core_axis_name="c", subcore_axis_name="s")

    @functools.partial(
        pl.kernel,
        out_type=jax.ShapeDtypeStruct((TOTAL_ROWS, Cp), jnp.float32),
        mesh=mesh,
        compiler_params=pltpu.CompilerParams(use_tc_tiling_on_sc=False),
        scratch_types=[
            pltpu.VMEM((GCHUNK,), jnp.int32),
            pltpu.VMEM((GCHUNK, Cp), jnp.float32),
            pltpu.SemaphoreType.DMA,
        ],
    )
    def gk(tab_hbm, ids_hbm, out_hbm, idx_v, rows_v, sem):
        wid = lax.axis_index("s") * SC_CORES + lax.axis_index("c")
        base = wid * ROWS_PER_W

        @pl.loop(0, NCHUNK)
        def _(ci):
            off = base + ci * GCHUNK
            pltpu.sync_copy(ids_hbm.at[pl.ds(off, GCHUNK)], idx_v)
            pltpu.async_copy(tab_hbm.at[idx_v], rows_v, sem).wait()
            pltpu.sync_copy(rows_v, out_hbm.at[pl.ds(off, GCHUNK)])

    return gk(table, ids)


def _gather_rows(xT, idx, Cp):
    C = xT.shape[2]
    if Cp != C:
        table = jnp.pad(xT, ((0, 0), (0, 0), (0, Cp - C)))
    else:
        table = xT
    g = _sc_gather_call(table.reshape(B * N, Cp), idx.reshape(TOTAL_ROWS), Cp)
    return g.reshape(B, K, N, Cp)


def _conv_body(g_ref, x_ref, w_ref, mx_ref, mn_ref, p_ref, *, C):
    xb = x_ref[0]
    xb16 = xb.astype(jnp.bfloat16)
    w = w_ref[...]
    mx = mn = s1 = s2 = None
    for t in range(K):
        gk = g_ref[0, t][:, :C]
        d16 = (gk - xb).astype(jnp.bfloat16)
        e = jnp.concatenate([d16, xb16], axis=1)
        h = lax.dot_general(e, w, (((1,), (0,)), ((), ())),
                            preferred_element_type=jnp.float32)
        if t == 0:
            mx = mn = s1 = h
            s2 = h * h
        else:
            mx = jnp.maximum(mx, h)
            mn = jnp.minimum(mn, h)
            s1 = s1 + h
            s2 = s2 + h * h
    mx_ref[0] = mx
    mn_ref[0] = mn
    p_ref[0, 0] = jnp.concatenate(
        [jnp.sum(s1, axis=0, keepdims=True),
         jnp.sum(s2, axis=0, keepdims=True)], axis=0)


def _conv_reduce(g, xT, w2c):
    C = xT.shape[2]
    Cp = g.shape[3]
    O = w2c.shape[1]
    return pl.pallas_call(
        functools.partial(_conv_body, C=C),
        grid=(B, NB),
        in_specs=[
            pl.BlockSpec((1, K, RB, Cp), lambda b, r: (b, 0, r, 0)),
            pl.BlockSpec((1, RB, C), lambda b, r: (b, r, 0)),
            pl.BlockSpec((2 * C, O), lambda b, r: (0, 0)),
        ],
        out_specs=[
            pl.BlockSpec((1, RB, O), lambda b, r: (b, r, 0)),
            pl.BlockSpec((1, RB, O), lambda b, r: (b, r, 0)),
            pl.BlockSpec((1, 1, 2, O), lambda b, r: (b, r, 0, 0)),
        ],
        out_shape=[
            jax.ShapeDtypeStruct((B, N, O), jnp.float32),
            jax.ShapeDtypeStruct((B, N, O), jnp.float32),
            jax.ShapeDtypeStruct((B, NB, 2, O), jnp.float32),
        ],
    )(g, xT, w2c)


def _fin_body(p_ref, md_ref, *, cnt):
    p = jnp.sum(p_ref[...], axis=0)
    mean = p[0:1] / cnt
    var = p[1:2] / cnt - mean * mean
    md_ref[...] = jnp.concatenate([mean, jnp.sqrt(var + EPS)], axis=0)


def _finalize(part, cnt):
    O = part.shape[-1]
    G = part.shape[0] * part.shape[1]
    return pl.pallas_call(
        functools.partial(_fin_body, cnt=float(cnt)),
        in_specs=[pl.BlockSpec((G, 2, O), lambda: (0, 0, 0))],
        out_specs=pl.BlockSpec((2, O), lambda: (0, 0)),
        out_shape=jax.ShapeDtypeStruct((2, O), jnp.float32),
    )(part.reshape(G, 2, O))


def _combine_body(mx_ref, mn_ref, md_ref, g_ref, b_ref, o_ref):
    mean = md_ref[0:1]
    den = md_ref[1:2]
    g = g_ref[...]
    sel = jnp.where(g >= 0, mx_ref[0], mn_ref[0])
    o_ref[0] = _leaky((sel - mean) / den * g + b_ref[...])


def _combine(mx, mn, md, gamma, beta):
    O = mx.shape[2]
    return pl.pallas_call(
        _combine_body,
        grid=(B,),
        in_specs=[
            pl.BlockSpec((1, N, O), lambda b: (b, 0, 0)),
            pl.BlockSpec((1, N, O), lambda b: (b, 0, 0)),
            pl.BlockSpec((2, O), lambda b: (0, 0)),
            pl.BlockSpec((1, O), lambda b: (0, 0)),
            pl.BlockSpec((1, O), lambda b: (0, 0)),
        ],
        out_specs=pl.BlockSpec((1, N, O), lambda b: (b, 0, 0)),
        out_shape=jax.ShapeDtypeStruct((B, N, O), jnp.float32),
    )(mx, mn, md, gamma.reshape(1, O), beta.reshape(1, O))


def _edge_layer(xT, W, gamma, beta, Cp):
    C = xT.shape[2]
    idx = _knn(xT)
    g = _gather_rows(xT, idx, Cp)
    w2c = jnp.transpose(W).astype(jnp.bfloat16)
    mx, mn, part = _conv_reduce(g, xT, w2c)
    md = _finalize(part, B * N * K)
    return _combine(mx, mn, md, gamma, beta)


def _head_mm_body(x1_ref, x2_ref, x3_ref, x4_ref,
                  w1_ref, w2_ref, w3_ref, w4_ref, z_ref, p_ref):
    z = None
    for xr, wr in ((x1_ref, w1_ref), (x2_ref, w2_ref),
                   (x3_ref, w3_ref), (x4_ref, w4_ref)):
        d = lax.dot_general(wr[...], xr[0].astype(jnp.bfloat16),
                            (((1,), (1,)), ((), ())),
                            preferred_element_type=jnp.float32)
        z = d if z is None else z + d
    z_ref[0] = z
    s1 = jnp.sum(z, axis=1, keepdims=True)
    s2 = jnp.sum(z * z, axis=1, keepdims=True)
    p_ref[0] = jnp.concatenate([s1, s2], axis=1)


def _head_fin_body(p_ref, md_ref):
    p = jnp.sum(p_ref[...], axis=0)
    cnt = jnp.float32(B * N)
    mean = p[:, 0:1] / cnt
    var = p[:, 1:2] / cnt - mean * mean
    md_ref[...] = jnp.concatenate([mean, jnp.sqrt(var + EPS)], axis=1)


def _head_out_body(z_ref, md_ref, g_ref, b_ref, feat_ref, pool_ref):
    mean = md_ref[:, 0:1]
    den = md_ref[:, 1:2]
    f = _leaky((z_ref[0] - mean) / den * g_ref[...] + b_ref[...])
    feat_ref[0] = f
    pool_ref[0] = jnp.max(f, axis=1, keepdims=True)


def _head(x1, x2, x3, x4, W5, gamma5, beta5):
    M = 1024
    w5 = W5.astype(jnp.bfloat16)
    ws = (w5[:, :64], w5[:, 64:128], w5[:, 128:256], w5[:, 256:512])
    z, part = pl.pallas_call(
        _head_mm_body,
        grid=(B,),
        in_specs=[pl.BlockSpec((1, N, 64), lambda b: (b, 0, 0)),
                  pl.BlockSpec((1, N, 64), lambda b: (b, 0, 0)),
                  pl.BlockSpec((1, N, 128), lambda b: (b, 0, 0)),
                  pl.BlockSpec((1, N, 256), lambda b: (b, 0, 0)),
                  pl.BlockSpec((M, 64), lambda b: (0, 0)),
                  pl.BlockSpec((M, 64), lambda b: (0, 0)),
                  pl.BlockSpec((M, 128), lambda b: (0, 0)),
                  pl.BlockSpec((M, 256), lambda b: (0, 0))],
        out_specs=[pl.BlockSpec((1, M, N), lambda b: (b, 0, 0)),
                   pl.BlockSpec((1, M, 2), lambda b: (b, 0, 0))],
        out_shape=[jax.ShapeDtypeStruct((B, M, N), jnp.float32),
                   jax.ShapeDtypeStruct((B, M, 2), jnp.float32)],
    )(x1, x2, x3, x4, *ws)
    md = pl.pallas_call(
        _head_fin_body,
        in_specs=[pl.BlockSpec((B, M, 2), lambda: (0, 0, 0))],
        out_specs=pl.BlockSpec((M, 2), lambda: (0, 0)),
        out_shape=jax.ShapeDtypeStruct((M, 2), jnp.float32),
    )(part)
    feat, pool = pl.pallas_call(
        _head_out_body,
        grid=(B,),
        in_specs=[pl.BlockSpec((1, M, N), lambda b: (b, 0, 0)),
                  pl.BlockSpec((M, 2), lambda b: (0, 0)),
                  pl.BlockSpec((M, 1), lambda b: (0, 0)),
                  pl.BlockSpec((M, 1), lambda b: (0, 0))],
        out_specs=[pl.BlockSpec((1, M, N), lambda b: (b, 0, 0)),
                   pl.BlockSpec((1, M, 1), lambda b: (b, 0, 0))],
        out_shape=[jax.ShapeDtypeStruct((B, M, N), jnp.float32),
                   jax.ShapeDtypeStruct((B, M, 1), jnp.float32)],
    )(z, md, gamma5.reshape(M, 1), beta5.reshape(M, 1))
    return pool.reshape(B, M), feat


def kernel(x, W1, W2, W3, W4, W5, gamma1, beta1, gamma2, beta2, gamma3,
           beta3, gamma4, beta4, gamma5, beta5):
    xT = jnp.transpose(x, (0, 2, 1))
    x1 = _edge_layer(xT, W1, gamma1, beta1, Cp=8)
    x2 = _edge_layer(x1, W2, gamma2, beta2, Cp=64)
    x3 = _edge_layer(x2, W3, gamma3, beta3, Cp=64)
    x4 = _edge_layer(x3, W4, gamma4, beta4, Cp=128)
    return _head(x1, x2, x3, x4, W5, gamma5, beta5)
